# baseline (speedup 1.0000x reference)
import functools

import jax
import jax.numpy as jnp
from jax import lax
from jax.experimental import pallas as pl
from jax.experimental.pallas import tpu as pltpu
from jax.experimental.pallas import tpu_sc as plsc

N = 10000
E = 320000
D = 128
NCLS = 16

NC = 2
NS = 16
NW = NC * NS

CH = 128
RB = 2
BB = 8
NB = 10
CPT = NB * BB
E_PAD = NW * CPT * CH
N_PAD = N + 112

_MESH = plsc.VectorSubcoreMesh(core_axis_name="core", subcore_axis_name="subcore")


@functools.partial(
    pl.kernel,
    out_type=jax.ShapeDtypeStruct((NC, 2, N_PAD), jnp.float32),
    mesh=_MESH,
    scratch_types=[
        pltpu.VMEM((CPT, CH), jnp.int32),
        pltpu.VMEM((CPT, CH), jnp.int32),
        pltpu.VMEM((CH,), jnp.float32),
        pltpu.VMEM_SHARED((N_PAD,), jnp.float32),
        pltpu.VMEM_SHARED((N_PAD,), jnp.float32),
    ],
)
def _sc_degrees(src_hbm, dst_hbm, ones_hbm, z1_hbm, deg_out,
                src_v, dst_v, ones_v, og_sh, ig_sh):
    c = lax.axis_index("core")
    s = lax.axis_index("subcore")
    wid = s * NC + c

    @pl.when(s == 0)
    def _():
        pltpu.sync_copy(z1_hbm, og_sh)
        pltpu.sync_copy(z1_hbm, ig_sh)

    pltpu.sync_copy(src_hbm.at[wid], src_v)
    pltpu.sync_copy(dst_hbm.at[wid], dst_v)
    pltpu.sync_copy(ones_hbm, ones_v)
    plsc.subcore_barrier()

    @pl.loop(0, CPT)
    def _(j):
        pltpu.sync_copy(ones_v, og_sh.at[src_v.at[j]], add=True)
        pltpu.sync_copy(ones_v, ig_sh.at[dst_v.at[j]], add=True)

    plsc.subcore_barrier()

    @pl.when(s == 0)
    def _():
        pltpu.sync_copy(og_sh, deg_out.at[c, 0])
        pltpu.sync_copy(ig_sh, deg_out.at[c, 1])


def _tc_mm_body(x_ref, w1_ref, xw_ref):
    xw_ref[...] = jnp.dot(x_ref[...], w1_ref[...],
                          preferred_element_type=jnp.float32)


_tc_mm = pl.pallas_call(
    _tc_mm_body,
    out_shape=jax.ShapeDtypeStruct((N, D), jnp.float32),
)


def _tc_prep_body(xw_ref, degp_ref, h_ref, on_ref, inn_ref):
    deg = degp_ref[0] + degp_ref[1]
    on = lax.rsqrt(jnp.maximum(deg[0], 1.0))
    inn = lax.rsqrt(jnp.maximum(deg[1], 1.0))
    on_ref[0, :] = on
    inn_ref[0, :] = inn
    h_ref[pl.ds(0, N), :] = xw_ref[...] * on[:N, None]
    h_ref[pl.ds(N, N_PAD - N), :] = jnp.zeros((N_PAD - N, D), jnp.float32)


_tc_prep = pl.pallas_call(
    _tc_prep_body,
    out_shape=(
        jax.ShapeDtypeStruct((N_PAD, D), jnp.float32),
        jax.ShapeDtypeStruct((1, N_PAD), jnp.float32),
        jax.ShapeDtypeStruct((1, N_PAD), jnp.float32),
    ),
)


@functools.partial(
    pl.kernel,
    out_type=(
        jax.ShapeDtypeStruct((NC, N_PAD, D), jnp.float32),
        jax.ShapeDtypeStruct((NC, N_PAD), jnp.float32),
    ),
    mesh=_MESH,
    scratch_types=[
        pltpu.VMEM((CPT, CH), jnp.int32),
        pltpu.VMEM((2, BB, CH), jnp.int32),
        pltpu.VMEM((CH, D), jnp.float32),
        pltpu.VMEM((CH, D), jnp.float32),
        pltpu.VMEM((CH,), jnp.float32),
        pltpu.VMEM_SHARED((N_PAD, D), jnp.float32),
        pltpu.VMEM_SHARED((N_PAD,), jnp.float32),
        [pltpu.SemaphoreType.DMA] * 2,
        [pltpu.SemaphoreType.DMA] * 2,
    ],
)
def _sc_edges(h_hbm, src_hbm, dst_hbm, innorm_hbm, zmat_hbm, z1_hbm,
              agg_out, c_out, src_v, dst_b, r0, r1, val_v,
              acc_sh, c_sh, gsem, bsem):
    c = lax.axis_index("core")
    s = lax.axis_index("subcore")
    wid = s * NC + c
    rows = (r0, r1)

    rpt0 = N_PAD // NS
    pltpu.sync_copy(zmat_hbm.at[pl.ds(s * rpt0, rpt0)],
                    acc_sh.at[pl.ds(s * rpt0, rpt0)])

    @pl.when(s == 0)
    def _():
        pltpu.sync_copy(z1_hbm, c_sh)

    pltpu.sync_copy(src_hbm.at[wid], src_v)
    for p in range(2):
        pltpu.async_copy(dst_hbm.at[wid, p], dst_b.at[p], bsem[p])
    plsc.subcore_barrier()

    for b in range(RB):
        pltpu.async_copy(h_hbm.at[src_v.at[b]], rows[b], gsem[b])

    def block_body(blk, p):
        pltpu.make_async_copy(dst_hbm.at[wid, blk], dst_b.at[p],
                              bsem[p]).wait()

        for i in range(BB):
            j = blk * BB + i
            q = i % 2
            pltpu.make_async_copy(h_hbm.at[src_v.at[j]], rows[q],
                                  gsem[q]).wait()
            pltpu.sync_copy(rows[q], acc_sh.at[dst_b.at[p, i]], add=True)

            @pl.when(j + RB < CPT)
            def _():
                pltpu.async_copy(h_hbm.at[src_v.at[j + RB]], rows[q], gsem[q])

            pltpu.sync_copy(innorm_hbm.at[dst_b.at[p, i]], val_v)
            pltpu.sync_copy(val_v, c_sh.at[src_v.at[j]], add=True)

        @pl.when(blk + 2 < NB)
        def _():
            pltpu.async_copy(dst_hbm.at[wid, blk + 2], dst_b.at[p], bsem[p])

    @pl.loop(0, NB, step=2)
    def _(kb):
        for p in range(2):
            block_body(kb + p, p)

    plsc.subcore_barrier()
    rpt = N_PAD // NS
    pltpu.sync_copy(acc_sh.at[pl.ds(s * rpt, rpt)],
                    agg_out.at[c, pl.ds(s * rpt, rpt)])

    @pl.when(s == 0)
    def _():
        pltpu.sync_copy(c_sh, c_out.at[c])


def _tc_final_body(aggp_ref, cp_ref, on_ref, inn_ref, b1_ref, w2_ref, b2_ref,
                   out_ref):
    agg = aggp_ref[0, pl.ds(0, N), :] + aggp_ref[1, pl.ds(0, N), :]
    cvec = cp_ref[0, pl.ds(0, N)] + cp_ref[1, pl.ds(0, N)]
    inn = inn_ref[0, pl.ds(0, N)]
    on = on_ref[0, pl.ds(0, N)]
    h1 = jnp.maximum(agg * inn[:, None] + b1_ref[0, :][None, :], 0.0)
    wn = on * cvec
    s = jnp.dot(wn[None, :], h1, preferred_element_type=jnp.float32)
    out = jnp.dot(s, w2_ref[...], preferred_element_type=jnp.float32)
    out_ref[...] = out * (1.0 / N) + b2_ref[0, :][None, :]


_tc_final = pl.pallas_call(
    _tc_final_body,
    out_shape=jax.ShapeDtypeStruct((1, NCLS), jnp.float32),
)


def kernel(in_feat, edge_index, W1, b1, W2, b2):
    src = edge_index[0]
    dst = edge_index[1]
    pad = E_PAD - E
    padv = N + (jnp.arange(pad, dtype=jnp.int32) % (N_PAD - N))
    src3 = jnp.concatenate([src, padv]).reshape(NW, CPT, CH)
    dst3 = jnp.concatenate([dst, padv]).reshape(NW, CPT, CH)
    dst4 = dst3.reshape(NW, NB, BB, CH)

    ones = jnp.ones((CH,), jnp.float32)
    z1 = jnp.zeros((N_PAD,), jnp.float32)
    zmat = jnp.zeros((N_PAD, D), jnp.float32)

    xw = _tc_mm(in_feat, W1)
    degp = _sc_degrees(src3, dst3, ones, z1)
    h, on2, inn2 = _tc_prep(xw, degp)
    aggp, cp = _sc_edges(h, src3, dst4, inn2.reshape(N_PAD), zmat, z1)
    out = _tc_final(aggp, cp, on2, inn2, b1.reshape(1, D),
                    W2, b2.reshape(1, NCLS))
    return out


# speedup vs baseline: 1.0121x; 1.0121x over previous
import functools

import jax
import jax.numpy as jnp
from jax import lax
from jax.experimental import pallas as pl
from jax.experimental.pallas import tpu as pltpu
from jax.experimental.pallas import tpu_sc as plsc

N = 10000
E = 320000
D = 128
NCLS = 16

NC = 2
NS = 16
NW = NC * NS

CH = 128
RB = 2
BB = 8
NB = 10
CPT = NB * BB
E_PAD = NW * CPT * CH
N_PAD = N + 112

_MESH = plsc.VectorSubcoreMesh(core_axis_name="core", subcore_axis_name="subcore")


@functools.partial(
    pl.kernel,
    out_type=jax.ShapeDtypeStruct((NC, 2, N_PAD), jnp.float32),
    mesh=_MESH,
    scratch_types=[
        pltpu.VMEM((CPT, CH), jnp.int32),
        pltpu.VMEM((CPT, CH), jnp.int32),
        pltpu.VMEM((CH,), jnp.float32),
        pltpu.VMEM_SHARED((N_PAD,), jnp.float32),
        pltpu.VMEM_SHARED((N_PAD,), jnp.float32),
    ],
)
def _sc_degrees(src_hbm, dst_hbm, ones_hbm, z1_hbm, deg_out,
                src_v, dst_v, ones_v, og_sh, ig_sh):
    c = lax.axis_index("core")
    s = lax.axis_index("subcore")
    wid = s * NC + c

    @pl.when(s == 0)
    def _():
        pltpu.sync_copy(z1_hbm, og_sh)
        pltpu.sync_copy(z1_hbm, ig_sh)

    pltpu.sync_copy(src_hbm.at[wid], src_v)
    pltpu.sync_copy(dst_hbm.at[wid], dst_v)
    pltpu.sync_copy(ones_hbm, ones_v)
    plsc.subcore_barrier()

    @pl.loop(0, CPT)
    def _(j):
        pltpu.sync_copy(ones_v, og_sh.at[src_v.at[j]], add=True)
        pltpu.sync_copy(ones_v, ig_sh.at[dst_v.at[j]], add=True)

    plsc.subcore_barrier()

    @pl.when(s == 0)
    def _():
        pltpu.sync_copy(og_sh, deg_out.at[c, 0])
        pltpu.sync_copy(ig_sh, deg_out.at[c, 1])


def _tc_prep_body(x_ref, w1_ref, degp_ref, h_ref, on_ref, inn_ref):
    deg = degp_ref[0] + degp_ref[1]
    on = lax.rsqrt(jnp.maximum(deg[0], 1.0))
    inn = lax.rsqrt(jnp.maximum(deg[1], 1.0))
    on_ref[0, :] = on
    inn_ref[0, :] = inn
    h = jnp.dot(x_ref[...], w1_ref[...], preferred_element_type=jnp.float32)
    h_ref[pl.ds(0, N), :] = h * on[:N, None]
    h_ref[pl.ds(N, N_PAD - N), :] = jnp.zeros((N_PAD - N, D), jnp.float32)


_tc_prep = pl.pallas_call(
    _tc_prep_body,
    out_shape=(
        jax.ShapeDtypeStruct((N_PAD, D), jnp.float32),
        jax.ShapeDtypeStruct((1, N_PAD), jnp.float32),
        jax.ShapeDtypeStruct((1, N_PAD), jnp.float32),
    ),
)


@functools.partial(
    pl.kernel,
    out_type=(
        jax.ShapeDtypeStruct((NC, N_PAD, D), jnp.float32),
        jax.ShapeDtypeStruct((NC, N_PAD), jnp.float32),
    ),
    mesh=_MESH,
    scratch_types=[
        pltpu.VMEM((CPT, CH), jnp.int32),
        pltpu.VMEM((2, BB, CH), jnp.int32),
        pltpu.VMEM((CH, D), jnp.float32),
        pltpu.VMEM((CH, D), jnp.float32),
        pltpu.VMEM((CH,), jnp.float32),
        pltpu.VMEM_SHARED((N_PAD, D), jnp.float32),
        pltpu.VMEM_SHARED((N_PAD,), jnp.float32),
        [pltpu.SemaphoreType.DMA] * 2,
        [pltpu.SemaphoreType.DMA] * 2,
    ],
)
def _sc_edges(h_hbm, src_hbm, dst_hbm, innorm_hbm, zmat_hbm, z1_hbm,
              agg_out, c_out, src_v, dst_b, r0, r1, val_v,
              acc_sh, c_sh, gsem, bsem):
    c = lax.axis_index("core")
    s = lax.axis_index("subcore")
    wid = s * NC + c
    rows = (r0, r1)

    rpt0 = N_PAD // NS
    pltpu.sync_copy(zmat_hbm.at[pl.ds(s * rpt0, rpt0)],
                    acc_sh.at[pl.ds(s * rpt0, rpt0)])

    @pl.when(s == 0)
    def _():
        pltpu.sync_copy(z1_hbm, c_sh)

    pltpu.sync_copy(src_hbm.at[wid], src_v)
    for p in range(2):
        pltpu.async_copy(dst_hbm.at[wid, p], dst_b.at[p], bsem[p])
    plsc.subcore_barrier()

    for b in range(RB):
        pltpu.async_copy(h_hbm.at[src_v.at[b]], rows[b], gsem[b])

    def block_body(blk, p):
        pltpu.make_async_copy(dst_hbm.at[wid, blk], dst_b.at[p],
                              bsem[p]).wait()

        for i in range(BB):
            j = blk * BB + i
            q = i % 2
            pltpu.make_async_copy(h_hbm.at[src_v.at[j]], rows[q],
                                  gsem[q]).wait()
            pltpu.sync_copy(rows[q], acc_sh.at[dst_b.at[p, i]], add=True)

            @pl.when(j + RB < CPT)
            def _():
                pltpu.async_copy(h_hbm.at[src_v.at[j + RB]], rows[q], gsem[q])

            pltpu.sync_copy(innorm_hbm.at[dst_b.at[p, i]], val_v)
            pltpu.sync_copy(val_v, c_sh.at[src_v.at[j]], add=True)

        @pl.when(blk + 2 < NB)
        def _():
            pltpu.async_copy(dst_hbm.at[wid, blk + 2], dst_b.at[p], bsem[p])

    @pl.loop(0, NB, step=2)
    def _(kb):
        for p in range(2):
            block_body(kb + p, p)

    plsc.subcore_barrier()
    rpt = N_PAD // NS
    pltpu.sync_copy(acc_sh.at[pl.ds(s * rpt, rpt)],
                    agg_out.at[c, pl.ds(s * rpt, rpt)])

    @pl.when(s == 0)
    def _():
        pltpu.sync_copy(c_sh, c_out.at[c])


def _tc_final_body(aggp_ref, cp_ref, on_ref, inn_ref, b1_ref, w2_ref, b2_ref,
                   out_ref):
    agg = aggp_ref[0, pl.ds(0, N), :] + aggp_ref[1, pl.ds(0, N), :]
    cvec = cp_ref[0, pl.ds(0, N)] + cp_ref[1, pl.ds(0, N)]
    inn = inn_ref[0, pl.ds(0, N)]
    on = on_ref[0, pl.ds(0, N)]
    h1 = jnp.maximum(agg * inn[:, None] + b1_ref[0, :][None, :], 0.0)
    wn = on * cvec
    s = jnp.dot(wn[None, :], h1, preferred_element_type=jnp.float32)
    out = jnp.dot(s, w2_ref[...], preferred_element_type=jnp.float32)
    out_ref[...] = out * (1.0 / N) + b2_ref[0, :][None, :]


_tc_final = pl.pallas_call(
    _tc_final_body,
    out_shape=jax.ShapeDtypeStruct((1, NCLS), jnp.float32),
)


def kernel(in_feat, edge_index, W1, b1, W2, b2):
    src = edge_index[0]
    dst = edge_index[1]
    pad = E_PAD - E
    padv = N + (jnp.arange(pad, dtype=jnp.int32) % (N_PAD - N))
    src3 = jnp.concatenate([src, padv]).reshape(NW, CPT, CH)
    dst3 = jnp.concatenate([dst, padv]).reshape(NW, CPT, CH)
    dst4 = dst3.reshape(NW, NB, BB, CH)

    ones = jnp.ones((CH,), jnp.float32)
    z1 = jnp.zeros((N_PAD,), jnp.float32)
    zmat = jnp.zeros((N_PAD, D), jnp.float32)

    degp = _sc_degrees(src3, dst3, ones, z1)
    h, on2, inn2 = _tc_prep(in_feat, W1, degp)
    aggp, cp = _sc_edges(h, src3, dst4, inn2.reshape(N_PAD), zmat, z1)
    out = _tc_final(aggp, cp, on2, inn2, b1.reshape(1, D),
                    W2, b2.reshape(1, NCLS))
    return out


# speedup vs baseline: 1.0422x; 1.0298x over previous
import functools

import jax
import jax.numpy as jnp
from jax import lax
from jax.experimental import pallas as pl
from jax.experimental.pallas import tpu as pltpu
from jax.experimental.pallas import tpu_sc as plsc

N = 10000
E = 320000
D = 128
NCLS = 16

NC = 2
NS = 16
NW = NC * NS

CH = 128
RB = 2
BB = 8
NB = 10
CPT = NB * BB
E_PAD = NW * CPT * CH
N_PAD = N + 112

_MESH = plsc.VectorSubcoreMesh(core_axis_name="core", subcore_axis_name="subcore")


@functools.partial(
    pl.kernel,
    out_type=jax.ShapeDtypeStruct((NW, 2, N_PAD), jnp.float32),
    mesh=_MESH,
    scratch_types=[
        pltpu.VMEM((CPT, CH), jnp.int32),
        pltpu.VMEM((CPT, CH), jnp.int32),
        pltpu.VMEM((N_PAD,), jnp.float32),
        pltpu.VMEM((N_PAD,), jnp.float32),
    ],
    compiler_params=pltpu.CompilerParams(needs_layout_passes=False),
)
def _sc_degrees(src_hbm, dst_hbm, z1_hbm, deg_out,
                src_v, dst_v, og_v, ig_v):
    c = lax.axis_index("core")
    s = lax.axis_index("subcore")
    wid = s * NC + c

    pltpu.sync_copy(z1_hbm, og_v)
    pltpu.sync_copy(z1_hbm, ig_v)
    pltpu.sync_copy(src_hbm.at[wid], src_v)
    pltpu.sync_copy(dst_hbm.at[wid], dst_v)

    ones16 = jnp.ones((16,), jnp.float32)

    @pl.loop(0, CPT)
    def _(j):
        for k in range(CH // 16):
            sl = pl.ds(k * 16, 16)
            plsc.addupdate_scatter(og_v, [src_v[j, sl]], ones16)
            plsc.addupdate_scatter(ig_v, [dst_v[j, sl]], ones16)

    pltpu.sync_copy(og_v, deg_out.at[wid, 0])
    pltpu.sync_copy(ig_v, deg_out.at[wid, 1])


def _tc_prep_body(x_ref, w1_ref, degp_ref, h_ref, on_ref, inn_ref):
    deg = jnp.sum(degp_ref[...], axis=0)
    on = lax.rsqrt(jnp.maximum(deg[0], 1.0))
    inn = lax.rsqrt(jnp.maximum(deg[1], 1.0))
    on_ref[0, :] = on
    inn_ref[0, :] = inn
    h = jnp.dot(x_ref[...], w1_ref[...], preferred_element_type=jnp.float32)
    h_ref[pl.ds(0, N), :] = h * on[:N, None]
    h_ref[pl.ds(N, N_PAD - N), :] = jnp.zeros((N_PAD - N, D), jnp.float32)


_tc_prep = pl.pallas_call(
    _tc_prep_body,
    out_shape=(
        jax.ShapeDtypeStruct((N_PAD, D), jnp.float32),
        jax.ShapeDtypeStruct((1, N_PAD), jnp.float32),
        jax.ShapeDtypeStruct((1, N_PAD), jnp.float32),
    ),
)


@functools.partial(
    pl.kernel,
    out_type=(
        jax.ShapeDtypeStruct((NC, N_PAD, D), jnp.float32),
        jax.ShapeDtypeStruct((NC, N_PAD), jnp.float32),
    ),
    mesh=_MESH,
    scratch_types=[
        pltpu.VMEM((CPT, CH), jnp.int32),
        pltpu.VMEM((2, BB, CH), jnp.int32),
        pltpu.VMEM((CH, D), jnp.float32),
        pltpu.VMEM((CH, D), jnp.float32),
        pltpu.VMEM((CH,), jnp.float32),
        pltpu.VMEM_SHARED((N_PAD, D), jnp.float32),
        pltpu.VMEM_SHARED((N_PAD,), jnp.float32),
        [pltpu.SemaphoreType.DMA] * 2,
        [pltpu.SemaphoreType.DMA] * 2,
    ],
)
def _sc_edges(h_hbm, src_hbm, dst_hbm, innorm_hbm, zmat_hbm, z1_hbm,
              agg_out, c_out, src_v, dst_b, r0, r1, val_v,
              acc_sh, c_sh, gsem, bsem):
    c = lax.axis_index("core")
    s = lax.axis_index("subcore")
    wid = s * NC + c
    rows = (r0, r1)

    rpt0 = N_PAD // NS
    pltpu.sync_copy(zmat_hbm.at[pl.ds(s * rpt0, rpt0)],
                    acc_sh.at[pl.ds(s * rpt0, rpt0)])

    @pl.when(s == 0)
    def _():
        pltpu.sync_copy(z1_hbm, c_sh)

    pltpu.sync_copy(src_hbm.at[wid], src_v)
    for p in range(2):
        pltpu.async_copy(dst_hbm.at[wid, p], dst_b.at[p], bsem[p])
    plsc.subcore_barrier()

    for b in range(RB):
        pltpu.async_copy(h_hbm.at[src_v.at[b]], rows[b], gsem[b])

    def block_body(blk, p):
        pltpu.make_async_copy(dst_hbm.at[wid, blk], dst_b.at[p],
                              bsem[p]).wait()

        for i in range(BB):
            j = blk * BB + i
            q = i % 2
            pltpu.make_async_copy(h_hbm.at[src_v.at[j]], rows[q],
                                  gsem[q]).wait()
            pltpu.sync_copy(rows[q], acc_sh.at[dst_b.at[p, i]], add=True)

            @pl.when(j + RB < CPT)
            def _():
                pltpu.async_copy(h_hbm.at[src_v.at[j + RB]], rows[q], gsem[q])

            pltpu.sync_copy(innorm_hbm.at[dst_b.at[p, i]], val_v)
            pltpu.sync_copy(val_v, c_sh.at[src_v.at[j]], add=True)

        @pl.when(blk + 2 < NB)
        def _():
            pltpu.async_copy(dst_hbm.at[wid, blk + 2], dst_b.at[p], bsem[p])

    @pl.loop(0, NB, step=2)
    def _(kb):
        for p in range(2):
            block_body(kb + p, p)

    plsc.subcore_barrier()
    rpt = N_PAD // NS
    pltpu.sync_copy(acc_sh.at[pl.ds(s * rpt, rpt)],
                    agg_out.at[c, pl.ds(s * rpt, rpt)])

    @pl.when(s == 0)
    def _():
        pltpu.sync_copy(c_sh, c_out.at[c])


def _tc_final_body(aggp_ref, cp_ref, on_ref, inn_ref, b1_ref, w2_ref, b2_ref,
                   out_ref):
    agg = aggp_ref[0, pl.ds(0, N), :] + aggp_ref[1, pl.ds(0, N), :]
    cvec = cp_ref[0, pl.ds(0, N)] + cp_ref[1, pl.ds(0, N)]
    inn = inn_ref[0, pl.ds(0, N)]
    on = on_ref[0, pl.ds(0, N)]
    h1 = jnp.maximum(agg * inn[:, None] + b1_ref[0, :][None, :], 0.0)
    wn = on * cvec
    s = jnp.dot(wn[None, :], h1, preferred_element_type=jnp.float32)
    out = jnp.dot(s, w2_ref[...], preferred_element_type=jnp.float32)
    out_ref[...] = out * (1.0 / N) + b2_ref[0, :][None, :]


_tc_final = pl.pallas_call(
    _tc_final_body,
    out_shape=jax.ShapeDtypeStruct((1, NCLS), jnp.float32),
)


def kernel(in_feat, edge_index, W1, b1, W2, b2):
    src = edge_index[0]
    dst = edge_index[1]
    pad = E_PAD - E
    padv = N + (jnp.arange(pad, dtype=jnp.int32) % (N_PAD - N))
    src3 = jnp.concatenate([src, padv]).reshape(NW, CPT, CH)
    dst3 = jnp.concatenate([dst, padv]).reshape(NW, CPT, CH)
    dst4 = dst3.reshape(NW, NB, BB, CH)

    z1 = jnp.zeros((N_PAD,), jnp.float32)
    zmat = jnp.zeros((N_PAD, D), jnp.float32)

    degp = _sc_degrees(src3, dst3, z1)
    h, on2, inn2 = _tc_prep(in_feat, W1, degp)
    aggp, cp = _sc_edges(h, src3, dst4, inn2.reshape(N_PAD), zmat, z1)
    out = _tc_final(aggp, cp, on2, inn2, b1.reshape(1, D),
                    W2, b2.reshape(1, NCLS))
    return out
